# baseline (speedup 1.0000x reference)
import functools
import math

import jax
import jax.numpy as jnp
from jax import lax
from jax.experimental import pallas as pl
from jax.experimental.pallas import tpu as pltpu
from jax.experimental.pallas import tpu_sc as plsc

H = 16
K = 8

_NC = 2
_NS = 16
_L = 16
_NW = _NC * _NS


def _qkv_body(x_ref, wq_ref, wk_ref, wv_ref, bq_ref, bk_ref, bv_ref,
              q_ref, k_ref, v_ref):
    xb = x_ref[...]
    q_ref[...] = jnp.dot(xb, wq_ref[...], preferred_element_type=jnp.float32) + bq_ref[...]
    k_ref[...] = jnp.dot(xb, wk_ref[...], preferred_element_type=jnp.float32) + bk_ref[...]
    v_ref[...] = jnp.dot(xb, wv_ref[...], preferred_element_type=jnp.float32) + bv_ref[...]


def _head_attn(q, k, v, scale):
    s = jax.lax.dot_general(
        q, k, (((1,), (1,)), ((), ())), preferred_element_type=jnp.float32
    ) * scale
    half = s.shape[1] // 2
    s1 = s[:, :half]
    s2 = s[:, half:]
    v1 = v[:half]
    v2 = v[half:]

    m1 = jnp.max(s1, axis=1, keepdims=True)
    e1 = jnp.exp(s1 - m1)
    bs1 = jnp.sum(e1, axis=1, keepdims=True)
    o1 = jnp.dot(e1, v1, preferred_element_type=jnp.float32)
    out1 = o1 * (1.0 / bs1)

    m2 = jnp.max(s2, axis=1, keepdims=True)
    mnew = jnp.maximum(m1, m2)
    delta = jnp.where(m1 == mnew, 0.0, m1 - mnew)
    ed = jnp.exp(delta)
    e2 = jnp.exp(s2 - mnew)
    bs2 = jnp.sum(e2, axis=1, keepdims=True)
    resc = ed * bs1
    sum2 = resc + bs2
    acc = resc * out1
    o2 = jnp.dot(e2, v2, preferred_element_type=jnp.float32) + acc
    return o2 * (1.0 / sum2)


def _attn_body(q_ref, k_ref, v_ref, o_ref, k0_s, k1_s, v0_s, v1_s, *, scale, dh):
    j = pl.program_id(1)

    @pl.when(j == 0)
    def _stash():
        kp = k_ref[...]
        vp = v_ref[...]
        k0_s[...] = kp[:, :dh]
        k1_s[...] = kp[:, dh:]
        v0_s[...] = vp[:, :dh]
        v1_s[...] = vp[:, dh:]

    qp = q_ref[...]
    c0 = _head_attn(qp[:, :dh], k0_s[...], v0_s[...], scale)
    c1 = _head_attn(qp[:, dh:], k1_s[...], v1_s[...], scale)
    o_ref[...] = jnp.concatenate([c0, c1], axis=1)


def _sc_gather_mix(neurons, idx_flat, wrep, S, D):
    TPW = S // _NW
    CH = 4
    NCH = TPW // CH
    RPC = CH * K
    DL = D // _L
    idx2d = idx_flat.reshape(S * K // RPC, RPC)
    mesh = plsc.VectorSubcoreMesh(core_axis_name="c", subcore_axis_name="s")

    import functools as _ft

    @_ft.partial(
        pl.kernel, mesh=mesh,
        out_type=jax.ShapeDtypeStruct((S, D), jnp.float32),
        scratch_types=[
            pltpu.VMEM((NCH, RPC), jnp.int32),
            pltpu.VMEM((CH * K, _L), jnp.float32),
            pltpu.VMEM((CH * K, _L), jnp.float32),
            pltpu.VMEM((RPC, D), jnp.float32),
            pltpu.VMEM((RPC, D), jnp.float32),
            pltpu.VMEM((CH, D), jnp.float32),
            pltpu.SemaphoreType.DMA,
            pltpu.SemaphoreType.DMA,
        ],
    )
    def k(neurons_hbm, idx_hbm, w_hbm, out_hbm,
          idx_v, w0_v, w1_v, r0_v, r1_v, out_v, sem0, sem1):
        wid = lax.axis_index("s") * _NC + lax.axis_index("c")
        tok0 = wid * TPW
        pltpu.sync_copy(idx_hbm.at[pl.ds(wid * NCH, NCH)], idx_v)

        def compute_store(c, rows_v, w_v):
            for t in range(CH):
                ws = [w_v[t * K + kk, :] for kk in range(K)]

                def dbody(i, _, ws=ws, t=t, rows_v=rows_v):
                    dd = pl.multiple_of(i * _L, _L)
                    prods = [ws[kk] * rows_v[t * K + kk, pl.ds(dd, _L)]
                             for kk in range(K)]
                    while len(prods) > 1:
                        prods = [a + b for a, b in zip(prods[::2], prods[1::2])]
                    out_v[t, pl.ds(dd, _L)] = prods[0]
                    return _
                lax.fori_loop(0, DL, dbody, None, unroll=16)
            pltpu.sync_copy(out_v, out_hbm.at[pl.ds(tok0 + c * CH, CH)])

        pltpu.async_copy(neurons_hbm.at[idx_v.at[0]], r0_v, sem0)
        pltpu.sync_copy(w_hbm.at[pl.ds(tok0 * K, CH * K)], w0_v)

        def pair(p, _):
            c0 = 2 * p
            pltpu.async_copy(neurons_hbm.at[idx_v.at[c0 + 1]], r1_v, sem1)
            pltpu.sync_copy(
                w_hbm.at[pl.ds((tok0 + (c0 + 1) * CH) * K, CH * K)], w1_v)
            pltpu.make_async_copy(
                neurons_hbm.at[idx_v.at[c0]], r0_v, sem0).wait()
            compute_store(c0, r0_v, w0_v)

            @pl.when(c0 + 2 < NCH)
            def _():
                pltpu.async_copy(
                    neurons_hbm.at[idx_v.at[c0 + 2]], r0_v, sem0)
                pltpu.sync_copy(
                    w_hbm.at[pl.ds((tok0 + (c0 + 2) * CH) * K, CH * K)], w0_v)
            pltpu.make_async_copy(
                neurons_hbm.at[idx_v.at[c0 + 1]], r1_v, sem1).wait()
            compute_store(c0 + 1, r1_v, w1_v)
            return _
        lax.fori_loop(0, NCH // 2, pair, None)

    return k(neurons, idx2d, wrep)


def _router_body(x_ref, c_ref, wp_ref, bp_ref, n_ref,
                 idx_ref, tw_ref, sel_ref, *, n_neurons):
    xb = x_ref[...]
    cb = c_ref[...]
    comb = jnp.concatenate([xb, cb], axis=1)
    logits = (
        jnp.dot(comb, wp_ref[...], preferred_element_type=jnp.float32)
        + bp_ref[...]
    )
    m = jnp.max(logits, axis=1, keepdims=True)
    e = jnp.exp(logits - m)
    w = e / jnp.sum(e, axis=1, keepdims=True)
    token_s = jax.lax.dot_general(
        xb, n_ref[...], (((1,), (1,)), ((), ())),
        preferred_element_type=jnp.float32,
    )
    ctx_s = jax.lax.dot_general(
        cb, n_ref[...], (((1,), (1,)), ((), ())),
        preferred_element_type=jnp.float32,
    )
    scores = w[:, 0:1] * token_s + w[:, 1:2] * ctx_s

    bt = scores.shape[0]
    iota_n = jax.lax.broadcasted_iota(jnp.int32, (bt, n_neurons), 1)
    iota_k = jax.lax.broadcasted_iota(jnp.int32, (bt, K), 1)
    s = scores
    tv = jnp.zeros((bt, K), dtype=jnp.float32)
    ti = jnp.zeros((bt, K), dtype=jnp.int32)
    picks = []
    for k in range(K):
        mk = jnp.max(s, axis=1, keepdims=True)
        ak = jnp.min(
            jnp.where(s == mk, iota_n, n_neurons), axis=1, keepdims=True
        )
        picks.append(ak)
        tv = jnp.where(iota_k == k, mk, tv)
        ti = jnp.where(iota_k == k, ak, ti)
        s = jnp.where(iota_n == ak, -jnp.inf, s)

    ew = jnp.exp(tv - tv[:, 0:1])
    tw = ew / jnp.sum(ew, axis=1, keepdims=True)

    idx_ref[...] = ti
    tw_ref[...] = tw

    sel = jnp.zeros((bt, n_neurons), dtype=jnp.float32)
    for k in range(K):
        sel = sel + jnp.where(iota_n == picks[k], tw[:, k:k + 1], 0.0)
    sel_ref[...] = sel


def kernel(x, neurons, Wq, bq, Wk, bk, Wv, bv, Wp, bp):
    Bsz, S, D = x.shape
    dh = D // H
    n_neurons = neurons.shape[0]
    x2 = x.reshape(S, D)

    BT = min(256, S)
    nblk = S // BT

    q2, k2, v2 = pl.pallas_call(
        _qkv_body,
        grid=(nblk,),
        in_specs=[
            pl.BlockSpec((BT, D), lambda j: (j, 0)),
            pl.BlockSpec((D, D), lambda j: (0, 0)),
            pl.BlockSpec((D, D), lambda j: (0, 0)),
            pl.BlockSpec((D, D), lambda j: (0, 0)),
            pl.BlockSpec((1, D), lambda j: (0, 0)),
            pl.BlockSpec((1, D), lambda j: (0, 0)),
            pl.BlockSpec((1, D), lambda j: (0, 0)),
        ],
        out_specs=[
            pl.BlockSpec((BT, D), lambda j: (j, 0)),
            pl.BlockSpec((BT, D), lambda j: (j, 0)),
            pl.BlockSpec((BT, D), lambda j: (j, 0)),
        ],
        out_shape=[
            jax.ShapeDtypeStruct((S, D), jnp.float32),
            jax.ShapeDtypeStruct((S, D), jnp.float32),
            jax.ShapeDtypeStruct((S, D), jnp.float32),
        ],
    )(x2, Wq, Wk, Wv, bq.reshape(1, D), bk.reshape(1, D), bv.reshape(1, D))

    hp = H // 2
    context = pl.pallas_call(
        functools.partial(_attn_body, scale=1.0 / math.sqrt(dh), dh=dh),
        grid=(hp, nblk),
        in_specs=[
            pl.BlockSpec((BT, 2 * dh), lambda h, j: (j, h)),
            pl.BlockSpec((S, 2 * dh), lambda h, j: (0, h)),
            pl.BlockSpec((S, 2 * dh), lambda h, j: (0, h)),
        ],
        out_specs=pl.BlockSpec((BT, 2 * dh), lambda h, j: (j, h)),
        out_shape=jax.ShapeDtypeStruct((S, D), jnp.float32),
        scratch_shapes=[
            pltpu.VMEM((S, dh), jnp.float32),
            pltpu.VMEM((S, dh), jnp.float32),
            pltpu.VMEM((S, dh), jnp.float32),
            pltpu.VMEM((S, dh), jnp.float32),
        ],
    )(q2, k2, v2)

    topk_idx, topk_w, sel = pl.pallas_call(
        functools.partial(_router_body, n_neurons=n_neurons),
        grid=(nblk,),
        in_specs=[
            pl.BlockSpec((BT, D), lambda j: (j, 0)),
            pl.BlockSpec((BT, D), lambda j: (j, 0)),
            pl.BlockSpec((2 * D, 2), lambda j: (0, 0)),
            pl.BlockSpec((1, 2), lambda j: (0, 0)),
            pl.BlockSpec((n_neurons, D), lambda j: (0, 0)),
        ],
        out_specs=[
            pl.BlockSpec((BT, K), lambda j: (j, 0)),
            pl.BlockSpec((BT, K), lambda j: (j, 0)),
            pl.BlockSpec((BT, n_neurons), lambda j: (j, 0)),
        ],
        out_shape=[
            jax.ShapeDtypeStruct((S, K), jnp.int32),
            jax.ShapeDtypeStruct((S, K), jnp.float32),
            jax.ShapeDtypeStruct((S, n_neurons), jnp.float32),
        ],
    )(x2, context, Wp, bp.reshape(1, 2), neurons)

    wrep = jnp.broadcast_to(topk_w.reshape(S * K, 1), (S * K, _L))
    out = _sc_gather_mix(neurons, topk_idx.reshape(S * K), wrep, S, D)

    return (
        out.reshape(Bsz, S, D),
        topk_idx.reshape(Bsz, S, K),
        topk_w.reshape(Bsz, S, K),
        sel.reshape(Bsz, S, n_neurons),
    )


# speedup vs baseline: 1.1139x; 1.1139x over previous
import functools
import math

import jax
import jax.numpy as jnp
from jax import lax
from jax.experimental import pallas as pl
from jax.experimental.pallas import tpu as pltpu
from jax.experimental.pallas import tpu_sc as plsc

H = 16
K = 8

_NC = 2
_NS = 16
_L = 16
_NW = _NC * _NS


def _qkv_body(x_ref, wq_ref, wk_ref, wv_ref, bq_ref, bk_ref, bv_ref,
              q_ref, k_ref, v_ref):
    xb = x_ref[...]
    q_ref[...] = jnp.dot(xb, wq_ref[...], preferred_element_type=jnp.float32) + bq_ref[...]
    k_ref[...] = jnp.dot(xb, wk_ref[...], preferred_element_type=jnp.float32) + bk_ref[...]
    v_ref[...] = jnp.dot(xb, wv_ref[...], preferred_element_type=jnp.float32) + bv_ref[...]


def _head_attn(q, k, v, scale):
    s = jax.lax.dot_general(
        q, k, (((1,), (1,)), ((), ())), preferred_element_type=jnp.float32
    ) * scale
    half = s.shape[1] // 2
    s1 = s[:, :half]
    s2 = s[:, half:]
    v1 = v[:half]
    v2 = v[half:]

    m1 = jnp.max(s1, axis=1, keepdims=True)
    e1 = jnp.exp(s1 - m1)
    bs1 = jnp.sum(e1, axis=1, keepdims=True)
    o1 = jnp.dot(e1, v1, preferred_element_type=jnp.float32)
    out1 = o1 * (1.0 / bs1)

    m2 = jnp.max(s2, axis=1, keepdims=True)
    mnew = jnp.maximum(m1, m2)
    delta = jnp.where(m1 == mnew, 0.0, m1 - mnew)
    ed = jnp.exp(delta)
    e2 = jnp.exp(s2 - mnew)
    bs2 = jnp.sum(e2, axis=1, keepdims=True)
    resc = ed * bs1
    sum2 = resc + bs2
    acc = resc * out1
    o2 = jnp.dot(e2, v2, preferred_element_type=jnp.float32) + acc
    return o2 * (1.0 / sum2)


def _attn_body(q_ref, k_ref, v_ref, o_ref, k0_s, k1_s, v0_s, v1_s, *, scale, dh):
    j = pl.program_id(1)

    @pl.when(j == 0)
    def _stash():
        kp = k_ref[...]
        vp = v_ref[...]
        k0_s[...] = kp[:, :dh]
        k1_s[...] = kp[:, dh:]
        v0_s[...] = vp[:, :dh]
        v1_s[...] = vp[:, dh:]

    qp = q_ref[...]
    c0 = _head_attn(qp[:, :dh], k0_s[...], v0_s[...], scale)
    c1 = _head_attn(qp[:, dh:], k1_s[...], v1_s[...], scale)
    o_ref[...] = jnp.concatenate([c0, c1], axis=1)


def _sc_gather_mix(neurons, idx_flat, wrep, S, D):
    TPW = S // _NW
    CH = 4
    NCH = TPW // CH
    RPC = CH * K
    DL = D // _L
    idx2d = idx_flat.reshape(S * K // RPC, RPC)
    mesh = plsc.VectorSubcoreMesh(core_axis_name="c", subcore_axis_name="s")

    import functools as _ft

    @_ft.partial(
        pl.kernel, mesh=mesh,
        out_type=jax.ShapeDtypeStruct((S, D), jnp.float32),
        scratch_types=[
            pltpu.VMEM((NCH, RPC), jnp.int32),
            pltpu.VMEM((CH * K, _L), jnp.float32),
            pltpu.VMEM((CH * K, _L), jnp.float32),
            pltpu.VMEM((RPC, D), jnp.float32),
            pltpu.VMEM((RPC, D), jnp.float32),
            pltpu.VMEM((CH, D), jnp.float32),
            pltpu.SemaphoreType.DMA,
            pltpu.SemaphoreType.DMA,
        ],
    )
    def k(neurons_hbm, idx_hbm, w_hbm, out_hbm,
          idx_v, w0_v, w1_v, r0_v, r1_v, out_v, sem0, sem1):
        wid = lax.axis_index("s") * _NC + lax.axis_index("c")
        tok0 = wid * TPW
        pltpu.sync_copy(idx_hbm.at[pl.ds(wid * NCH, NCH)], idx_v)

        def compute_store(c, rows_v, w_v):
            for t in range(CH):
                ws = [w_v[t * K + kk, :] for kk in range(K)]

                @plsc.parallel_loop(0, DL, step=1, unroll=4)
                def dbody(i, ws=ws, t=t, rows_v=rows_v):
                    dd = pl.multiple_of(i * _L, _L)
                    prods = [ws[kk] * rows_v[t * K + kk, pl.ds(dd, _L)]
                             for kk in range(K)]
                    while len(prods) > 1:
                        prods = [a + b for a, b in zip(prods[::2], prods[1::2])]
                    out_v[t, pl.ds(dd, _L)] = prods[0]
            pltpu.sync_copy(out_v, out_hbm.at[pl.ds(tok0 + c * CH, CH)])

        pltpu.async_copy(neurons_hbm.at[idx_v.at[0]], r0_v, sem0)
        pltpu.sync_copy(w_hbm.at[pl.ds(tok0 * K, CH * K)], w0_v)

        def pair(p, _):
            c0 = 2 * p
            pltpu.async_copy(neurons_hbm.at[idx_v.at[c0 + 1]], r1_v, sem1)
            pltpu.sync_copy(
                w_hbm.at[pl.ds((tok0 + (c0 + 1) * CH) * K, CH * K)], w1_v)
            pltpu.make_async_copy(
                neurons_hbm.at[idx_v.at[c0]], r0_v, sem0).wait()
            compute_store(c0, r0_v, w0_v)

            @pl.when(c0 + 2 < NCH)
            def _():
                pltpu.async_copy(
                    neurons_hbm.at[idx_v.at[c0 + 2]], r0_v, sem0)
                pltpu.sync_copy(
                    w_hbm.at[pl.ds((tok0 + (c0 + 2) * CH) * K, CH * K)], w0_v)
            pltpu.make_async_copy(
                neurons_hbm.at[idx_v.at[c0 + 1]], r1_v, sem1).wait()
            compute_store(c0 + 1, r1_v, w1_v)
            return _
        lax.fori_loop(0, NCH // 2, pair, None)

    return k(neurons, idx2d, wrep)


def _router_body(x_ref, c_ref, wp_ref, bp_ref, n_ref,
                 idx_ref, tw_ref, sel_ref, *, n_neurons):
    xb = x_ref[...]
    cb = c_ref[...]
    comb = jnp.concatenate([xb, cb], axis=1)
    logits = (
        jnp.dot(comb, wp_ref[...], preferred_element_type=jnp.float32)
        + bp_ref[...]
    )
    m = jnp.max(logits, axis=1, keepdims=True)
    e = jnp.exp(logits - m)
    w = e / jnp.sum(e, axis=1, keepdims=True)
    token_s = jax.lax.dot_general(
        xb, n_ref[...], (((1,), (1,)), ((), ())),
        preferred_element_type=jnp.float32,
    )
    ctx_s = jax.lax.dot_general(
        cb, n_ref[...], (((1,), (1,)), ((), ())),
        preferred_element_type=jnp.float32,
    )
    scores = w[:, 0:1] * token_s + w[:, 1:2] * ctx_s

    bt = scores.shape[0]
    iota_n = jax.lax.broadcasted_iota(jnp.int32, (bt, n_neurons), 1)
    iota_k = jax.lax.broadcasted_iota(jnp.int32, (bt, K), 1)
    s = scores
    tv = jnp.zeros((bt, K), dtype=jnp.float32)
    ti = jnp.zeros((bt, K), dtype=jnp.int32)
    picks = []
    for k in range(K):
        mk = jnp.max(s, axis=1, keepdims=True)
        ak = jnp.min(
            jnp.where(s == mk, iota_n, n_neurons), axis=1, keepdims=True
        )
        picks.append(ak)
        tv = jnp.where(iota_k == k, mk, tv)
        ti = jnp.where(iota_k == k, ak, ti)
        s = jnp.where(iota_n == ak, -jnp.inf, s)

    ew = jnp.exp(tv - tv[:, 0:1])
    tw = ew / jnp.sum(ew, axis=1, keepdims=True)

    idx_ref[...] = ti
    tw_ref[...] = tw

    sel = jnp.zeros((bt, n_neurons), dtype=jnp.float32)
    for k in range(K):
        sel = sel + jnp.where(iota_n == picks[k], tw[:, k:k + 1], 0.0)
    sel_ref[...] = sel


def kernel(x, neurons, Wq, bq, Wk, bk, Wv, bv, Wp, bp):
    Bsz, S, D = x.shape
    dh = D // H
    n_neurons = neurons.shape[0]
    x2 = x.reshape(S, D)

    BT = min(256, S)
    nblk = S // BT

    q2, k2, v2 = pl.pallas_call(
        _qkv_body,
        grid=(nblk,),
        in_specs=[
            pl.BlockSpec((BT, D), lambda j: (j, 0)),
            pl.BlockSpec((D, D), lambda j: (0, 0)),
            pl.BlockSpec((D, D), lambda j: (0, 0)),
            pl.BlockSpec((D, D), lambda j: (0, 0)),
            pl.BlockSpec((1, D), lambda j: (0, 0)),
            pl.BlockSpec((1, D), lambda j: (0, 0)),
            pl.BlockSpec((1, D), lambda j: (0, 0)),
        ],
        out_specs=[
            pl.BlockSpec((BT, D), lambda j: (j, 0)),
            pl.BlockSpec((BT, D), lambda j: (j, 0)),
            pl.BlockSpec((BT, D), lambda j: (j, 0)),
        ],
        out_shape=[
            jax.ShapeDtypeStruct((S, D), jnp.float32),
            jax.ShapeDtypeStruct((S, D), jnp.float32),
            jax.ShapeDtypeStruct((S, D), jnp.float32),
        ],
    )(x2, Wq, Wk, Wv, bq.reshape(1, D), bk.reshape(1, D), bv.reshape(1, D))

    hp = H // 2
    context = pl.pallas_call(
        functools.partial(_attn_body, scale=1.0 / math.sqrt(dh), dh=dh),
        grid=(hp, nblk),
        in_specs=[
            pl.BlockSpec((BT, 2 * dh), lambda h, j: (j, h)),
            pl.BlockSpec((S, 2 * dh), lambda h, j: (0, h)),
            pl.BlockSpec((S, 2 * dh), lambda h, j: (0, h)),
        ],
        out_specs=pl.BlockSpec((BT, 2 * dh), lambda h, j: (j, h)),
        out_shape=jax.ShapeDtypeStruct((S, D), jnp.float32),
        scratch_shapes=[
            pltpu.VMEM((S, dh), jnp.float32),
            pltpu.VMEM((S, dh), jnp.float32),
            pltpu.VMEM((S, dh), jnp.float32),
            pltpu.VMEM((S, dh), jnp.float32),
        ],
    )(q2, k2, v2)

    topk_idx, topk_w, sel = pl.pallas_call(
        functools.partial(_router_body, n_neurons=n_neurons),
        grid=(nblk,),
        in_specs=[
            pl.BlockSpec((BT, D), lambda j: (j, 0)),
            pl.BlockSpec((BT, D), lambda j: (j, 0)),
            pl.BlockSpec((2 * D, 2), lambda j: (0, 0)),
            pl.BlockSpec((1, 2), lambda j: (0, 0)),
            pl.BlockSpec((n_neurons, D), lambda j: (0, 0)),
        ],
        out_specs=[
            pl.BlockSpec((BT, K), lambda j: (j, 0)),
            pl.BlockSpec((BT, K), lambda j: (j, 0)),
            pl.BlockSpec((BT, n_neurons), lambda j: (j, 0)),
        ],
        out_shape=[
            jax.ShapeDtypeStruct((S, K), jnp.int32),
            jax.ShapeDtypeStruct((S, K), jnp.float32),
            jax.ShapeDtypeStruct((S, n_neurons), jnp.float32),
        ],
    )(x2, context, Wp, bp.reshape(1, 2), neurons)

    wrep = jnp.broadcast_to(topk_w.reshape(S * K, 1), (S * K, _L))
    out = _sc_gather_mix(neurons, topk_idx.reshape(S * K), wrep, S, D)

    return (
        out.reshape(Bsz, S, D),
        topk_idx.reshape(Bsz, S, K),
        topk_w.reshape(Bsz, S, K),
        sel.reshape(Bsz, S, n_neurons),
    )


# speedup vs baseline: 1.2505x; 1.1227x over previous
import functools
import math

import jax
import jax.numpy as jnp
from jax import lax
from jax.experimental import pallas as pl
from jax.experimental.pallas import tpu as pltpu
from jax.experimental.pallas import tpu_sc as plsc

H = 16
K = 8

_NC = 2
_NS = 16
_L = 16
_NW = _NC * _NS


def _qkv_body(x_ref, wq_ref, wk_ref, wv_ref, bq_ref, bk_ref, bv_ref,
              q_ref, k_ref, v_ref):
    xb = x_ref[...]
    q_ref[...] = jnp.dot(xb, wq_ref[...], preferred_element_type=jnp.float32) + bq_ref[...]
    k_ref[...] = jnp.dot(xb, wk_ref[...], preferred_element_type=jnp.float32) + bk_ref[...]
    v_ref[...] = jnp.dot(xb, wv_ref[...], preferred_element_type=jnp.float32) + bv_ref[...]


def _head_attn(q, k, v, scale):
    s = jax.lax.dot_general(
        q, k, (((1,), (1,)), ((), ())), preferred_element_type=jnp.float32
    ) * scale
    half = s.shape[1] // 2
    s1 = s[:, :half]
    s2 = s[:, half:]
    v1 = v[:half]
    v2 = v[half:]

    m1 = jnp.max(s1, axis=1, keepdims=True)
    e1 = jnp.exp(s1 - m1)
    bs1 = jnp.sum(e1, axis=1, keepdims=True)
    o1 = jnp.dot(e1, v1, preferred_element_type=jnp.float32)
    out1 = o1 * (1.0 / bs1)

    m2 = jnp.max(s2, axis=1, keepdims=True)
    mnew = jnp.maximum(m1, m2)
    delta = jnp.where(m1 == mnew, 0.0, m1 - mnew)
    ed = jnp.exp(delta)
    e2 = jnp.exp(s2 - mnew)
    bs2 = jnp.sum(e2, axis=1, keepdims=True)
    resc = ed * bs1
    sum2 = resc + bs2
    acc = resc * out1
    o2 = jnp.dot(e2, v2, preferred_element_type=jnp.float32) + acc
    return o2 * (1.0 / sum2)


def _attn_body(q_ref, k_ref, v_ref, o_ref, k0_s, k1_s, v0_s, v1_s, *, scale, dh):
    j = pl.program_id(1)

    @pl.when(j == 0)
    def _stash():
        kp = k_ref[...]
        vp = v_ref[...]
        k0_s[...] = kp[:, :dh]
        k1_s[...] = kp[:, dh:]
        v0_s[...] = vp[:, :dh]
        v1_s[...] = vp[:, dh:]

    qp = q_ref[...]
    c0 = _head_attn(qp[:, :dh], k0_s[...], v0_s[...], scale)
    c1 = _head_attn(qp[:, dh:], k1_s[...], v1_s[...], scale)
    o_ref[...] = jnp.concatenate([c0, c1], axis=1)


def _sc_gather_mix(neurons, idx_flat, wrep, S, D):
    TPW = S // _NW
    CH = 4
    NCH = TPW // CH
    RPC = CH * K
    DL = D // _L
    idx2d = idx_flat.reshape(S * K // RPC, RPC)
    mesh = plsc.VectorSubcoreMesh(core_axis_name="c", subcore_axis_name="s")

    import functools as _ft

    @_ft.partial(
        pl.kernel, mesh=mesh,
        out_type=jax.ShapeDtypeStruct((S, D), jnp.float32),
        scratch_types=[
            pltpu.VMEM((NCH, RPC), jnp.int32),
            pltpu.VMEM((CH * K, _L), jnp.float32),
            pltpu.VMEM((CH * K, _L), jnp.float32),
            pltpu.VMEM((RPC, D), jnp.float32),
            pltpu.VMEM((RPC, D), jnp.float32),
            pltpu.VMEM((CH, D), jnp.float32),
            pltpu.SemaphoreType.DMA,
            pltpu.SemaphoreType.DMA,
        ],
    )
    def k(neurons_hbm, idx_hbm, w_hbm, out_hbm,
          idx_v, w0_v, w1_v, r0_v, r1_v, out_v, sem0, sem1):
        wid = lax.axis_index("s") * _NC + lax.axis_index("c")
        tok0 = wid * TPW
        pltpu.sync_copy(idx_hbm.at[pl.ds(wid * NCH, NCH)], idx_v)

        def compute_store(c, rows_v, w_v):
            for t in range(CH):
                ws = [w_v[t * K + kk, :] for kk in range(K)]

                @plsc.parallel_loop(0, DL, step=1, unroll=4)
                def dbody(i, ws=ws, t=t, rows_v=rows_v):
                    dd = pl.multiple_of(i * _L, _L)
                    prods = [ws[kk] * rows_v[t * K + kk, pl.ds(dd, _L)]
                             for kk in range(K)]
                    while len(prods) > 1:
                        prods = [a + b for a, b in zip(prods[::2], prods[1::2])]
                    out_v[t, pl.ds(dd, _L)] = prods[0]
            pltpu.sync_copy(out_v, out_hbm.at[pl.ds(tok0 + c * CH, CH)])

        pltpu.async_copy(neurons_hbm.at[idx_v.at[0]], r0_v, sem0)
        pltpu.sync_copy(w_hbm.at[pl.ds(tok0 * K, CH * K)], w0_v)

        def pair(p, _):
            c0 = 2 * p
            pltpu.async_copy(neurons_hbm.at[idx_v.at[c0 + 1]], r1_v, sem1)
            pltpu.sync_copy(
                w_hbm.at[pl.ds((tok0 + (c0 + 1) * CH) * K, CH * K)], w1_v)
            pltpu.make_async_copy(
                neurons_hbm.at[idx_v.at[c0]], r0_v, sem0).wait()
            compute_store(c0, r0_v, w0_v)

            @pl.when(c0 + 2 < NCH)
            def _():
                pltpu.async_copy(
                    neurons_hbm.at[idx_v.at[c0 + 2]], r0_v, sem0)
                pltpu.sync_copy(
                    w_hbm.at[pl.ds((tok0 + (c0 + 2) * CH) * K, CH * K)], w0_v)
            pltpu.make_async_copy(
                neurons_hbm.at[idx_v.at[c0 + 1]], r1_v, sem1).wait()
            compute_store(c0 + 1, r1_v, w1_v)
            return _
        lax.fori_loop(0, NCH // 2, pair, None)

    return k(neurons, idx2d, wrep)


def _router_body(x_ref, c_ref, wp_ref, bp_ref, n_ref,
                 idx_ref, tw_ref, sel_ref, *, n_neurons):
    xb = x_ref[...]
    cb = c_ref[...]
    comb = jnp.concatenate([xb, cb], axis=1)
    logits = (
        jnp.dot(comb, wp_ref[...], preferred_element_type=jnp.float32)
        + bp_ref[...]
    )
    m = jnp.max(logits, axis=1, keepdims=True)
    e = jnp.exp(logits - m)
    w = e / jnp.sum(e, axis=1, keepdims=True)
    token_s = jax.lax.dot_general(
        xb, n_ref[...], (((1,), (1,)), ((), ())),
        preferred_element_type=jnp.float32,
    )
    ctx_s = jax.lax.dot_general(
        cb, n_ref[...], (((1,), (1,)), ((), ())),
        preferred_element_type=jnp.float32,
    )
    scores = w[:, 0:1] * token_s + w[:, 1:2] * ctx_s

    bt = scores.shape[0]
    iota_n = jax.lax.broadcasted_iota(jnp.int32, (bt, n_neurons), 1)
    iota_k = jax.lax.broadcasted_iota(jnp.int32, (bt, K), 1)
    s = scores
    tv = jnp.zeros((bt, K), dtype=jnp.float32)
    ti = jnp.zeros((bt, K), dtype=jnp.int32)
    picks = []
    for k in range(K):
        mk = jnp.max(s, axis=1, keepdims=True)
        ak = jnp.min(
            jnp.where(s == mk, iota_n, n_neurons), axis=1, keepdims=True
        )
        picks.append(ak)
        tv = jnp.where(iota_k == k, mk, tv)
        ti = jnp.where(iota_k == k, ak, ti)
        s = jnp.where(iota_n == ak, -jnp.inf, s)

    ew = jnp.exp(tv - tv[:, 0:1])
    tw = ew / jnp.sum(ew, axis=1, keepdims=True)

    idx_ref[...] = ti
    tw_ref[...] = tw

    sel = jnp.zeros((bt, n_neurons), dtype=jnp.float32)
    for k in range(K):
        sel = sel + jnp.where(iota_n == picks[k], tw[:, k:k + 1], 0.0)
    sel_ref[...] = sel


def kernel(x, neurons, Wq, bq, Wk, bk, Wv, bv, Wp, bp):
    Bsz, S, D = x.shape
    dh = D // H
    n_neurons = neurons.shape[0]
    x2 = x.reshape(S, D)

    BT = min(512, S)
    nblk = S // BT

    q2, k2, v2 = pl.pallas_call(
        _qkv_body,
        grid=(nblk,),
        in_specs=[
            pl.BlockSpec((BT, D), lambda j: (j, 0)),
            pl.BlockSpec((D, D), lambda j: (0, 0)),
            pl.BlockSpec((D, D), lambda j: (0, 0)),
            pl.BlockSpec((D, D), lambda j: (0, 0)),
            pl.BlockSpec((1, D), lambda j: (0, 0)),
            pl.BlockSpec((1, D), lambda j: (0, 0)),
            pl.BlockSpec((1, D), lambda j: (0, 0)),
        ],
        out_specs=[
            pl.BlockSpec((BT, D), lambda j: (j, 0)),
            pl.BlockSpec((BT, D), lambda j: (j, 0)),
            pl.BlockSpec((BT, D), lambda j: (j, 0)),
        ],
        out_shape=[
            jax.ShapeDtypeStruct((S, D), jnp.float32),
            jax.ShapeDtypeStruct((S, D), jnp.float32),
            jax.ShapeDtypeStruct((S, D), jnp.float32),
        ],
    )(x2, Wq, Wk, Wv, bq.reshape(1, D), bk.reshape(1, D), bv.reshape(1, D))

    hp = H // 2
    context = pl.pallas_call(
        functools.partial(_attn_body, scale=1.0 / math.sqrt(dh), dh=dh),
        grid=(hp, nblk),
        in_specs=[
            pl.BlockSpec((BT, 2 * dh), lambda h, j: (j, h)),
            pl.BlockSpec((S, 2 * dh), lambda h, j: (0, h)),
            pl.BlockSpec((S, 2 * dh), lambda h, j: (0, h)),
        ],
        out_specs=pl.BlockSpec((BT, 2 * dh), lambda h, j: (j, h)),
        out_shape=jax.ShapeDtypeStruct((S, D), jnp.float32),
        scratch_shapes=[
            pltpu.VMEM((S, dh), jnp.float32),
            pltpu.VMEM((S, dh), jnp.float32),
            pltpu.VMEM((S, dh), jnp.float32),
            pltpu.VMEM((S, dh), jnp.float32),
        ],
    )(q2, k2, v2)

    topk_idx, topk_w, sel = pl.pallas_call(
        functools.partial(_router_body, n_neurons=n_neurons),
        grid=(nblk,),
        in_specs=[
            pl.BlockSpec((BT, D), lambda j: (j, 0)),
            pl.BlockSpec((BT, D), lambda j: (j, 0)),
            pl.BlockSpec((2 * D, 2), lambda j: (0, 0)),
            pl.BlockSpec((1, 2), lambda j: (0, 0)),
            pl.BlockSpec((n_neurons, D), lambda j: (0, 0)),
        ],
        out_specs=[
            pl.BlockSpec((BT, K), lambda j: (j, 0)),
            pl.BlockSpec((BT, K), lambda j: (j, 0)),
            pl.BlockSpec((BT, n_neurons), lambda j: (j, 0)),
        ],
        out_shape=[
            jax.ShapeDtypeStruct((S, K), jnp.int32),
            jax.ShapeDtypeStruct((S, K), jnp.float32),
            jax.ShapeDtypeStruct((S, n_neurons), jnp.float32),
        ],
    )(x2, context, Wp, bp.reshape(1, 2), neurons)

    wrep = jnp.broadcast_to(topk_w.reshape(S * K, 1), (S * K, _L))
    out = _sc_gather_mix(neurons, topk_idx.reshape(S * K), wrep, S, D)

    return (
        out.reshape(Bsz, S, D),
        topk_idx.reshape(Bsz, S, K),
        topk_w.reshape(Bsz, S, K),
        sel.reshape(Bsz, S, n_neurons),
    )


# speedup vs baseline: 1.2728x; 1.0178x over previous
import functools
import math

import jax
import jax.numpy as jnp
from jax import lax
from jax.experimental import pallas as pl
from jax.experimental.pallas import tpu as pltpu
from jax.experimental.pallas import tpu_sc as plsc

H = 16
K = 8

_NC = 2
_NS = 16
_L = 16
_NW = _NC * _NS


def _qkv_body(x_ref, wq_ref, wk_ref, wv_ref, bq_ref, bk_ref, bv_ref,
              q_ref, k_ref, v_ref):
    xb = x_ref[...]
    q_ref[...] = jnp.dot(xb, wq_ref[...], preferred_element_type=jnp.float32) + bq_ref[...]
    k_ref[...] = jnp.dot(xb, wk_ref[...], preferred_element_type=jnp.float32) + bk_ref[...]
    v_ref[...] = jnp.dot(xb, wv_ref[...], preferred_element_type=jnp.float32) + bv_ref[...]


def _head_attn(q, k, v, scale):
    s = jax.lax.dot_general(
        q, k, (((1,), (1,)), ((), ())), preferred_element_type=jnp.float32
    ) * scale
    half = s.shape[1] // 2
    s1 = s[:, :half]
    s2 = s[:, half:]
    v1 = v[:half]
    v2 = v[half:]

    m1 = jnp.max(s1, axis=1, keepdims=True)
    e1 = jnp.exp(s1 - m1)
    bs1 = jnp.sum(e1, axis=1, keepdims=True)
    o1 = jnp.dot(e1, v1, preferred_element_type=jnp.float32)
    out1 = o1 * (1.0 / bs1)

    m2 = jnp.max(s2, axis=1, keepdims=True)
    mnew = jnp.maximum(m1, m2)
    delta = jnp.where(m1 == mnew, 0.0, m1 - mnew)
    ed = jnp.exp(delta)
    e2 = jnp.exp(s2 - mnew)
    bs2 = jnp.sum(e2, axis=1, keepdims=True)
    resc = ed * bs1
    sum2 = resc + bs2
    acc = resc * out1
    o2 = jnp.dot(e2, v2, preferred_element_type=jnp.float32) + acc
    return o2 * (1.0 / sum2)


def _attn_body(q_ref, k_ref, v_ref, o_ref, k0_s, k1_s, v0_s, v1_s, *, scale, dh):
    j = pl.program_id(1)

    @pl.when(j == 0)
    def _stash():
        kp = k_ref[...]
        vp = v_ref[...]
        k0_s[...] = kp[:, :dh]
        k1_s[...] = kp[:, dh:]
        v0_s[...] = vp[:, :dh]
        v1_s[...] = vp[:, dh:]

    qp = q_ref[...]
    c0 = _head_attn(qp[:, :dh], k0_s[...], v0_s[...], scale)
    c1 = _head_attn(qp[:, dh:], k1_s[...], v1_s[...], scale)
    o_ref[...] = jnp.concatenate([c0, c1], axis=1)


def _sc_gather_mix(neurons, idx_flat, wrep, S, D):
    TPW = S // _NW
    CH = 4
    NCH = TPW // CH
    RPC = CH * K
    DL = D // _L
    idx2d = idx_flat.reshape(S * K // RPC, RPC)
    mesh = plsc.VectorSubcoreMesh(core_axis_name="c", subcore_axis_name="s")

    import functools as _ft

    @_ft.partial(
        pl.kernel, mesh=mesh,
        out_type=jax.ShapeDtypeStruct((S, D), jnp.float32),
        scratch_types=[
            pltpu.VMEM((NCH, RPC), jnp.int32),
            pltpu.VMEM((CH * K, _L), jnp.float32),
            pltpu.VMEM((CH * K, _L), jnp.float32),
            pltpu.VMEM((RPC, D), jnp.float32),
            pltpu.VMEM((RPC, D), jnp.float32),
            pltpu.VMEM((CH, D), jnp.float32),
            pltpu.SemaphoreType.DMA,
            pltpu.SemaphoreType.DMA,
        ],
    )
    def k(neurons_hbm, idx_hbm, w_hbm, out_hbm,
          idx_v, w0_v, w1_v, r0_v, r1_v, out_v, sem0, sem1):
        wid = lax.axis_index("s") * _NC + lax.axis_index("c")
        tok0 = wid * TPW
        pltpu.sync_copy(idx_hbm.at[pl.ds(wid * NCH, NCH)], idx_v)

        def compute_store(c, rows_v, w_v):
            for t in range(CH):
                ws = [w_v[t * K + kk, :] for kk in range(K)]

                @plsc.parallel_loop(0, DL, step=1, unroll=4)
                def dbody(i, ws=ws, t=t, rows_v=rows_v):
                    dd = pl.multiple_of(i * _L, _L)
                    prods = [ws[kk] * rows_v[t * K + kk, pl.ds(dd, _L)]
                             for kk in range(K)]
                    while len(prods) > 1:
                        prods = [a + b for a, b in zip(prods[::2], prods[1::2])]
                    out_v[t, pl.ds(dd, _L)] = prods[0]
            pltpu.sync_copy(out_v, out_hbm.at[pl.ds(tok0 + c * CH, CH)])

        pltpu.async_copy(neurons_hbm.at[idx_v.at[0]], r0_v, sem0)
        pltpu.sync_copy(w_hbm.at[pl.ds(tok0 * K, CH * K)], w0_v)

        def pair(p, _):
            c0 = 2 * p
            pltpu.async_copy(neurons_hbm.at[idx_v.at[c0 + 1]], r1_v, sem1)
            pltpu.sync_copy(
                w_hbm.at[pl.ds((tok0 + (c0 + 1) * CH) * K, CH * K)], w1_v)
            pltpu.make_async_copy(
                neurons_hbm.at[idx_v.at[c0]], r0_v, sem0).wait()
            compute_store(c0, r0_v, w0_v)

            @pl.when(c0 + 2 < NCH)
            def _():
                pltpu.async_copy(
                    neurons_hbm.at[idx_v.at[c0 + 2]], r0_v, sem0)
                pltpu.sync_copy(
                    w_hbm.at[pl.ds((tok0 + (c0 + 2) * CH) * K, CH * K)], w0_v)
            pltpu.make_async_copy(
                neurons_hbm.at[idx_v.at[c0 + 1]], r1_v, sem1).wait()
            compute_store(c0 + 1, r1_v, w1_v)
            return _
        lax.fori_loop(0, NCH // 2, pair, None)

    return k(neurons, idx2d, wrep)


def _router_body(x_ref, c_ref, wp_ref, bp_ref, n_ref,
                 idx_ref, tw_ref, sel_ref, *, n_neurons):
    xb = x_ref[...]
    cb = c_ref[...]
    comb = jnp.concatenate([xb, cb], axis=1)
    logits = (
        jnp.dot(comb, wp_ref[...], preferred_element_type=jnp.float32)
        + bp_ref[...]
    )
    m = jnp.max(logits, axis=1, keepdims=True)
    e = jnp.exp(logits - m)
    w = e / jnp.sum(e, axis=1, keepdims=True)
    token_s = jax.lax.dot_general(
        xb, n_ref[...], (((1,), (1,)), ((), ())),
        preferred_element_type=jnp.float32,
    )
    ctx_s = jax.lax.dot_general(
        cb, n_ref[...], (((1,), (1,)), ((), ())),
        preferred_element_type=jnp.float32,
    )
    scores = w[:, 0:1] * token_s + w[:, 1:2] * ctx_s

    bt = scores.shape[0]
    iota_n = jax.lax.broadcasted_iota(jnp.int32, (bt, n_neurons), 1)
    iota_k = jax.lax.broadcasted_iota(jnp.int32, (bt, K), 1)
    s = scores
    tv = jnp.zeros((bt, K), dtype=jnp.float32)
    ti = jnp.zeros((bt, K), dtype=jnp.int32)
    picks = []
    for k in range(K):
        mk = jnp.max(s, axis=1, keepdims=True)
        ak = jnp.min(
            jnp.where(s == mk, iota_n, n_neurons), axis=1, keepdims=True
        )
        picks.append(ak)
        tv = jnp.where(iota_k == k, mk, tv)
        ti = jnp.where(iota_k == k, ak, ti)
        s = jnp.where(iota_n == ak, -jnp.inf, s)

    ew = jnp.exp(tv - tv[:, 0:1])
    tw = ew / jnp.sum(ew, axis=1, keepdims=True)

    idx_ref[...] = ti
    tw_ref[...] = tw

    sel = jnp.zeros((bt, n_neurons), dtype=jnp.float32)
    for k in range(K):
        sel = sel + jnp.where(iota_n == picks[k], tw[:, k:k + 1], 0.0)
    sel_ref[...] = sel


def kernel(x, neurons, Wq, bq, Wk, bk, Wv, bv, Wp, bp):
    Bsz, S, D = x.shape
    dh = D // H
    n_neurons = neurons.shape[0]
    x2 = x.reshape(S, D)

    BT = min(1024, S)
    nblk = S // BT

    q2, k2, v2 = pl.pallas_call(
        _qkv_body,
        grid=(nblk,),
        in_specs=[
            pl.BlockSpec((BT, D), lambda j: (j, 0)),
            pl.BlockSpec((D, D), lambda j: (0, 0)),
            pl.BlockSpec((D, D), lambda j: (0, 0)),
            pl.BlockSpec((D, D), lambda j: (0, 0)),
            pl.BlockSpec((1, D), lambda j: (0, 0)),
            pl.BlockSpec((1, D), lambda j: (0, 0)),
            pl.BlockSpec((1, D), lambda j: (0, 0)),
        ],
        out_specs=[
            pl.BlockSpec((BT, D), lambda j: (j, 0)),
            pl.BlockSpec((BT, D), lambda j: (j, 0)),
            pl.BlockSpec((BT, D), lambda j: (j, 0)),
        ],
        out_shape=[
            jax.ShapeDtypeStruct((S, D), jnp.float32),
            jax.ShapeDtypeStruct((S, D), jnp.float32),
            jax.ShapeDtypeStruct((S, D), jnp.float32),
        ],
    )(x2, Wq, Wk, Wv, bq.reshape(1, D), bk.reshape(1, D), bv.reshape(1, D))

    hp = H // 2
    context = pl.pallas_call(
        functools.partial(_attn_body, scale=1.0 / math.sqrt(dh), dh=dh),
        grid=(hp, nblk),
        in_specs=[
            pl.BlockSpec((BT, 2 * dh), lambda h, j: (j, h)),
            pl.BlockSpec((S, 2 * dh), lambda h, j: (0, h)),
            pl.BlockSpec((S, 2 * dh), lambda h, j: (0, h)),
        ],
        out_specs=pl.BlockSpec((BT, 2 * dh), lambda h, j: (j, h)),
        out_shape=jax.ShapeDtypeStruct((S, D), jnp.float32),
        scratch_shapes=[
            pltpu.VMEM((S, dh), jnp.float32),
            pltpu.VMEM((S, dh), jnp.float32),
            pltpu.VMEM((S, dh), jnp.float32),
            pltpu.VMEM((S, dh), jnp.float32),
        ],
    )(q2, k2, v2)

    topk_idx, topk_w, sel = pl.pallas_call(
        functools.partial(_router_body, n_neurons=n_neurons),
        grid=(nblk,),
        in_specs=[
            pl.BlockSpec((BT, D), lambda j: (j, 0)),
            pl.BlockSpec((BT, D), lambda j: (j, 0)),
            pl.BlockSpec((2 * D, 2), lambda j: (0, 0)),
            pl.BlockSpec((1, 2), lambda j: (0, 0)),
            pl.BlockSpec((n_neurons, D), lambda j: (0, 0)),
        ],
        out_specs=[
            pl.BlockSpec((BT, K), lambda j: (j, 0)),
            pl.BlockSpec((BT, K), lambda j: (j, 0)),
            pl.BlockSpec((BT, n_neurons), lambda j: (j, 0)),
        ],
        out_shape=[
            jax.ShapeDtypeStruct((S, K), jnp.int32),
            jax.ShapeDtypeStruct((S, K), jnp.float32),
            jax.ShapeDtypeStruct((S, n_neurons), jnp.float32),
        ],
    )(x2, context, Wp, bp.reshape(1, 2), neurons)

    wrep = jnp.broadcast_to(topk_w.reshape(S * K, 1), (S * K, _L))
    out = _sc_gather_mix(neurons, topk_idx.reshape(S * K), wrep, S, D)

    return (
        out.reshape(Bsz, S, D),
        topk_idx.reshape(Bsz, S, K),
        topk_w.reshape(Bsz, S, K),
        sel.reshape(Bsz, S, n_neurons),
    )
